# baseline (speedup 1.0000x reference)
import functools

import jax
import jax.numpy as jnp
from jax import lax
from jax.experimental import pallas as pl
from jax.experimental.pallas import tpu as pltpu
from jax.experimental.pallas import tpu_sc as plsc

B = 8
N = 1024
KNB = 20
CH = 64
CHP = 128

_DEFAULT = lax.Precision.DEFAULT
_HIGHEST = lax.Precision.HIGHEST


def _dot(a, b, dims, precision):
    return lax.dot_general(a, b, (dims, ((), ())), precision=precision,
                           preferred_element_type=jnp.float32)


def _pack_keys(p, iota):
    pi = lax.bitcast_convert_type(p, jnp.int32)
    pi = jnp.where(pi < 0, pi ^ jnp.int32(0x7FFFFFFF), pi)
    return (pi & jnp.int32(-1024)) | (jnp.int32(N - 1) - iota)


def _stage1_kernel(rows_ref, w1t_ref, hmax_ref, hsum_ref, hssq_ref, s_scr):
    rows = rows_ref[0]
    rb = rows.astype(jnp.bfloat16)
    s = _dot(rb, rb, (((1,), (1,))), _DEFAULT)
    xx = jnp.sum(rows * rows, axis=1)
    p = (-xx[:, None] + 2.0 * s) - xx[None, :]
    iota = lax.broadcasted_iota(jnp.int32, (N, N), 1)
    s_scr[...] = _pack_keys(p, iota)
    w1t = w1t_ref[...]
    h1 = rows.astype(jnp.bfloat16)
    r1 = rows - h1.astype(jnp.float32)
    h2 = r1.astype(jnp.bfloat16)
    r2 = r1 - h2.astype(jnp.float32)
    h3 = r2.astype(jnp.bfloat16)
    xcat = jnp.concatenate([h1, h2, h3], axis=1)

    def body(k, carry):
        m, hmax, hsum, hssq = carry
        kc = s_scr[...]
        mn = jnp.max(jnp.where(kc < m[:, None], kc, jnp.int32(-2 ** 31)),
                     axis=1)
        sel = kc == mn[:, None]
        xjp = _dot(sel.astype(jnp.bfloat16), xcat, (((1,), (0,))), _DEFAULT)
        xj = (xjp[:, 0:3] + xjp[:, 3:6]) + xjp[:, 6:9]
        f6 = jnp.concatenate([xj - rows, rows], axis=1)
        hk = _dot(f6, w1t, (((1,), (0,))), _DEFAULT)
        return (mn, jnp.maximum(hmax, hk), hsum + hk, hssq + hk * hk)

    _, hmax, hsum, hssq = lax.fori_loop(
        0, KNB, body,
        (jnp.full((N,), 2 ** 31 - 1, jnp.int32),
         jnp.full((N, CH), -3.0e38, jnp.float32),
         jnp.zeros((N, CH), jnp.float32),
         jnp.zeros((N, CH), jnp.float32)))
    hmax_ref[0] = hmax
    hsum_ref[0, 0] = jnp.sum(hsum, axis=0)
    hssq_ref[0, 0] = jnp.sum(hssq, axis=0)


def _stage1(rows, w1t):
    return pl.pallas_call(
        _stage1_kernel,
        grid=(B,),
        in_specs=[
            pl.BlockSpec((1, N, 3), lambda b: (b, 0, 0)),
            pl.BlockSpec((6, CH), lambda b: (0, 0)),
        ],
        out_specs=[
            pl.BlockSpec((1, N, CH), lambda b: (b, 0, 0)),
            pl.BlockSpec((1, 1, CH), lambda b: (b, 0, 0)),
            pl.BlockSpec((1, 1, CH), lambda b: (b, 0, 0)),
        ],
        out_shape=[
            jax.ShapeDtypeStruct((B, N, CH), jnp.float32),
            jax.ShapeDtypeStruct((B, 1, CH), jnp.float32),
            jax.ShapeDtypeStruct((B, 1, CH), jnp.float32),
        ],
        scratch_shapes=[pltpu.VMEM((N, N), jnp.int32)],
    )(rows, w1t)


def _stage1_fin_kernel(hmax_ref, hsum_ref, hssq_ref, g_ref, b_ref, x_ref):
    m = jnp.float32(B * N * KNB)
    mean = jnp.sum(hsum_ref[:, 0, :], axis=0) / m
    ex2 = jnp.sum(hssq_ref[:, 0, :], axis=0) / m
    var = ex2 - mean * mean
    xn = (hmax_ref[...] - mean[None, :]) / jnp.sqrt(var + 1e-5)[None, :]
    h = xn * g_ref[0][None, :] + b_ref[0][None, :]
    x_ref[...] = jnp.where(h > 0, h, 0.2 * h)


def _stage1_fin(hmax, hsum, hssq, g, b):
    return pl.pallas_call(
        _stage1_fin_kernel,
        out_shape=jax.ShapeDtypeStruct((B * N, CH), jnp.float32),
    )(hmax.reshape(B * N, CH), hsum, hssq, g.reshape(1, CH), b.reshape(1, CH))


def _knn_stage_kernel(rows_ref, at_ref, dt_ref, idx_ref, u_ref, v_ref, s_scr):
    b = pl.program_id(0)
    rows = rows_ref[0]
    rb = rows.astype(jnp.bfloat16)
    s = _dot(rb, rb, (((1,), (1,))), _DEFAULT)
    xx = jnp.sum(rows * rows, axis=1)
    p = (-xx[:, None] + 2.0 * s) - xx[None, :]
    iota = lax.broadcasted_iota(jnp.int32, (N, N), 1)
    s_scr[...] = _pack_keys(p, iota)

    def body(k, m):
        kc = s_scr[...]
        mn = jnp.max(jnp.where(kc < m[:, None], kc, jnp.int32(-2 ** 31)),
                     axis=1)
        a = jnp.int32(N - 1) - (mn & jnp.int32(N - 1))
        idx_ref[0, pl.ds(k, 1), :] = (a + b * N)[None, :]
        return mn

    lax.fori_loop(0, KNB, body, jnp.full((N,), 2 ** 31 - 1, jnp.int32))
    u_ref[0] = _dot(rows, at_ref[...], (((1,), (0,))), _HIGHEST)
    v_ref[0] = _dot(rows, dt_ref[...], (((1,), (0,))), _HIGHEST)


def _knn_stage(rows, a_t, d_t):
    c = rows.shape[-1]
    return pl.pallas_call(
        _knn_stage_kernel,
        grid=(B,),
        in_specs=[
            pl.BlockSpec((1, N, c), lambda b: (b, 0, 0)),
            pl.BlockSpec((c, CHP), lambda b: (0, 0)),
            pl.BlockSpec((c, CH), lambda b: (0, 0)),
        ],
        out_specs=[
            pl.BlockSpec((1, KNB, N), lambda b: (b, 0, 0)),
            pl.BlockSpec((1, N, CHP), lambda b: (b, 0, 0)),
            pl.BlockSpec((1, N, CH), lambda b: (b, 0, 0)),
        ],
        out_shape=[
            jax.ShapeDtypeStruct((B, KNB, N), jnp.int32),
            jax.ShapeDtypeStruct((B, N, CHP), jnp.float32),
            jax.ShapeDtypeStruct((B, N, CH), jnp.float32),
        ],
        scratch_shapes=[pltpu.VMEM((N, N), jnp.int32)],
    )(rows, a_t, d_t)


_NWORK = 32
_PER_W = (B * N) // _NWORK
_NCK = 32
_NCHUNK = _PER_W // _NCK


def _sc_reduce_body(u_hbm, idx_hbm, max_hbm, sum_hbm, ssq_hbm,
                    idx_v, rows_v, omax_v, osum_v, ossq_v, sem):
    wid = lax.axis_index("s") * 2 + lax.axis_index("c")
    g0 = wid * _PER_W
    bb = g0 // N
    i0 = g0 % N

    cps = [pltpu.async_copy(idx_hbm.at[bb, k, pl.ds(i0, _PER_W)],
                            idx_v.at[k], sem) for k in range(KNB)]
    for cp in cps:
        cp.wait()

    for ck in range(_NCHUNK):
        cps = [pltpu.async_copy(
            u_hbm.at[idx_v.at[k, pl.ds(ck * _NCK, _NCK)]],
            rows_v.at[k], sem) for k in range(KNB)]
        for cp in cps:
            cp.wait()

        def body(il, _):
            for ch in range(CH // 16):
                sl = pl.ds(ch * 16, 16)
                r = rows_v[0, il, sl]
                vmax = r
                vsum = r
                vssq = r * r
                for k in range(1, KNB):
                    r = rows_v[k, il, sl]
                    vmax = jnp.maximum(vmax, r)
                    vsum = vsum + r
                    vssq = vssq + r * r
                omax_v[il, sl] = vmax
                osum_v[il, sl] = vsum
                ossq_v[il, sl] = vssq
            return 0

        lax.fori_loop(0, _NCK, body, 0)
        base = g0 + ck * _NCK
        pltpu.sync_copy(omax_v, max_hbm.at[pl.ds(base, _NCK)])
        pltpu.sync_copy(osum_v, sum_hbm.at[pl.ds(base, _NCK)])
        pltpu.sync_copy(ossq_v, ssq_hbm.at[pl.ds(base, _NCK)])


def _sc_reduce(u_flat, idx):
    mesh = plsc.VectorSubcoreMesh(core_axis_name="c", subcore_axis_name="s")
    shp = jax.ShapeDtypeStruct((B * N, CH), jnp.float32)
    fn = pl.kernel(
        _sc_reduce_body,
        mesh=mesh,
        out_type=[shp, shp, shp],
        scratch_types=[
            pltpu.VMEM((KNB, _PER_W), jnp.int32),
            pltpu.VMEM((KNB, _NCK, CHP), jnp.float32),
            pltpu.VMEM((_NCK, CH), jnp.float32),
            pltpu.VMEM((_NCK, CH), jnp.float32),
            pltpu.VMEM((_NCK, CH), jnp.float32),
            pltpu.SemaphoreType.DMA,
        ],
    )
    return fn(u_flat, idx)


def _stats_kernel(usum_ref, ussq_ref, v_ref, g_ref, b_ref, sc_ref, sh_ref):
    usum = usum_ref[...]
    ussq = ussq_ref[...]
    v = v_ref[...]
    m = jnp.float32(B * N * KNB)
    sum_s = jnp.sum(usum, axis=0)
    sum_v = jnp.sum(v, axis=0)
    sum_s2 = jnp.sum(ussq, axis=0)
    sum_vs = jnp.sum(v * usum, axis=0)
    sum_v2 = jnp.sum(v * v, axis=0)
    mean = (sum_s + KNB * sum_v) / m
    ex2 = (sum_s2 + 2.0 * sum_vs + KNB * sum_v2) / m
    var = ex2 - mean * mean
    sc = g_ref[0] * lax.rsqrt(var + 1e-5)
    sh = b_ref[0] - mean * sc
    sc_ref[...] = jnp.broadcast_to(sc[None, :], (8, CH))
    sh_ref[...] = jnp.broadcast_to(sh[None, :], (8, CH))


def _bn_stats(usum, ussq, v_flat, g, b):
    return pl.pallas_call(
        _stats_kernel,
        out_shape=[jax.ShapeDtypeStruct((8, CH), jnp.float32),
                   jax.ShapeDtypeStruct((8, CH), jnp.float32)],
    )(usum, ussq, v_flat, g.reshape(1, CH), b.reshape(1, CH))


def _finalize_kernel(umax_ref, v_ref, sc_ref, sh_ref, x_ref):
    h = (umax_ref[...] + v_ref[...]) * sc_ref[0] + sh_ref[0]
    x_ref[...] = jnp.where(h > 0, h, 0.2 * h)


def _finalize(umax, v_flat, sc, sh):
    return pl.pallas_call(
        _finalize_kernel,
        out_shape=jax.ShapeDtypeStruct((B * N, CH), jnp.float32),
    )(umax, v_flat, sc, sh)


def _head_kernel(x1_ref, x2_ref, w3t_ref, g3_ref, b3_ref, wfct_ref, bfc_ref,
                 wp1t_ref, bp1_ref, wp2t_ref, bp2_ref, rep_ref, proj_ref):
    xcomb = jnp.concatenate([x1_ref[...], x2_ref[...]], axis=1)
    m3 = jnp.float32(B * N)
    mu = jnp.sum(xcomb, axis=0) / m3
    m2 = _dot(xcomb, xcomb, (((0,), (0,))), _HIGHEST) / m3
    w3t = w3t_ref[...]
    mean3 = mu @ w3t
    e2 = jnp.sum((m2 @ w3t) * w3t, axis=0)
    var3 = e2 - mean3 * mean3
    h3 = _dot(xcomb, w3t, (((1,), (0,))), _DEFAULT)
    h3max = jnp.max(h3.reshape(B, N, 512), axis=1)
    sc3 = g3_ref[0] * lax.rsqrt(var3 + 1e-5)
    xo = (h3max - mean3[None, :]) * sc3[None, :] + b3_ref[0][None, :]
    xo = jnp.where(xo > 0, xo, 0.2 * xo)
    rep = _dot(xo, wfct_ref[...], (((1,), (0,))), _DEFAULT) + bfc_ref[0][None, :]
    p1 = jnp.maximum(
        _dot(rep, wp1t_ref[...], (((1,), (0,))), _DEFAULT) + bp1_ref[0][None, :],
        0.0)
    proj = _dot(p1, wp2t_ref[...], (((1,), (0,))), _DEFAULT) + bp2_ref[0][None, :]
    rep_ref[...] = rep
    proj_ref[...] = proj


def _head(x1_flat, x2_flat, w3, g3, b3, wfc, bfc, wp1, bp1, wp2, bp2):
    return pl.pallas_call(
        _head_kernel,
        out_shape=[jax.ShapeDtypeStruct((B, 512), jnp.float32),
                   jax.ShapeDtypeStruct((B, 32), jnp.float32)],
    )(x1_flat, x2_flat, w3.T, g3.reshape(1, 512), b3.reshape(1, 512),
      wfc.T, bfc.reshape(1, 512), wp1.T, bp1.reshape(1, 256),
      wp2.T, bp2.reshape(1, 32))


def _edge_stage(rows, w, g, b, cin):
    a_t = jnp.zeros((cin, CHP), jnp.float32).at[:, :CH].set(w[:, :cin].T)
    d_t = (w[:, cin:] - w[:, :cin]).T
    idx, u, v = _knn_stage(rows, a_t, d_t)
    u_flat = u.reshape(B * N, CHP)
    v_flat = v.reshape(B * N, CH)
    umax, usum, ussq = _sc_reduce(u_flat, idx)
    sc, sh = _bn_stats(usum, ussq, v_flat, g, b)
    return _finalize(umax, v_flat, sc, sh)


def kernel(x, W1, g1, b1, W2, g2, b2, W3, g3, b3, Wfc, bfc, Wp1, bp1, Wp2, bp2):
    xt = jnp.transpose(x, (0, 2, 1))
    hmax, hsum, hssq = _stage1(xt, W1.T)
    x1 = _stage1_fin(hmax, hsum, hssq, g1, b1)
    x2 = _edge_stage(x1.reshape(B, N, CH), W2, g2, b2, CH)
    return _head(x1, x2, W3, g3, b3, Wfc, bfc, Wp1, bp1, Wp2, bp2)


# speedup vs baseline: 1.0170x; 1.0170x over previous
import functools

import jax
import jax.numpy as jnp
from jax import lax
from jax.experimental import pallas as pl
from jax.experimental.pallas import tpu as pltpu
from jax.experimental.pallas import tpu_sc as plsc

B = 8
N = 1024
KNB = 20
CH = 64
CHP = 128

_DEFAULT = lax.Precision.DEFAULT
_HIGHEST = lax.Precision.HIGHEST


def _dot(a, b, dims, precision):
    return lax.dot_general(a, b, (dims, ((), ())), precision=precision,
                           preferred_element_type=jnp.float32)


def _pack_keys(p, iota):
    pi = lax.bitcast_convert_type(p, jnp.int32)
    pi = jnp.where(pi < 0, pi ^ jnp.int32(0x7FFFFFFF), pi)
    return (pi & jnp.int32(-1024)) | (jnp.int32(N - 1) - iota)


def _stage1_kernel(rows_ref, w1t_ref, hmax_ref, hsum_ref, hssq_ref, s_scr):
    rows = rows_ref[0]
    rb = rows.astype(jnp.bfloat16)
    s = _dot(rb, rb, (((1,), (1,))), _DEFAULT)
    xx = jnp.sum(rows * rows, axis=1)
    p = (-xx[:, None] + 2.0 * s) - xx[None, :]
    iota = lax.broadcasted_iota(jnp.int32, (N, N), 1)
    s_scr[...] = _pack_keys(p, iota)
    w1t = w1t_ref[...]
    h1 = rows.astype(jnp.bfloat16)
    r1 = rows - h1.astype(jnp.float32)
    h2 = r1.astype(jnp.bfloat16)
    r2 = r1 - h2.astype(jnp.float32)
    h3 = r2.astype(jnp.bfloat16)
    xcat = jnp.concatenate([h1, h2, h3], axis=1)

    def body(k, carry):
        hmax, hsum, hssq = carry
        kc = s_scr[...]
        m = jnp.max(kc, axis=1)
        a = jnp.int32(N - 1) - (m & jnp.int32(N - 1))
        sel = iota == a[:, None]
        s_scr[...] = jnp.where(sel, jnp.int32(-2 ** 31 + 1), kc)
        xjp = _dot(sel.astype(jnp.bfloat16), xcat, (((1,), (0,))), _DEFAULT)
        xj = (xjp[:, 0:3] + xjp[:, 3:6]) + xjp[:, 6:9]
        f6 = jnp.concatenate([xj - rows, rows], axis=1)
        hk = _dot(f6, w1t, (((1,), (0,))), _DEFAULT)
        return (jnp.maximum(hmax, hk), hsum + hk, hssq + hk * hk)

    hmax, hsum, hssq = lax.fori_loop(
        0, KNB, body,
        (jnp.full((N, CH), -3.0e38, jnp.float32),
         jnp.zeros((N, CH), jnp.float32),
         jnp.zeros((N, CH), jnp.float32)))
    hmax_ref[0] = hmax
    hsum_ref[0, 0] = jnp.sum(hsum, axis=0)
    hssq_ref[0, 0] = jnp.sum(hssq, axis=0)


def _stage1(rows, w1t):
    return pl.pallas_call(
        _stage1_kernel,
        grid=(B,),
        in_specs=[
            pl.BlockSpec((1, N, 3), lambda b: (b, 0, 0)),
            pl.BlockSpec((6, CH), lambda b: (0, 0)),
        ],
        out_specs=[
            pl.BlockSpec((1, N, CH), lambda b: (b, 0, 0)),
            pl.BlockSpec((1, 1, CH), lambda b: (b, 0, 0)),
            pl.BlockSpec((1, 1, CH), lambda b: (b, 0, 0)),
        ],
        out_shape=[
            jax.ShapeDtypeStruct((B, N, CH), jnp.float32),
            jax.ShapeDtypeStruct((B, 1, CH), jnp.float32),
            jax.ShapeDtypeStruct((B, 1, CH), jnp.float32),
        ],
        scratch_shapes=[pltpu.VMEM((N, N), jnp.int32)],
    )(rows, w1t)


def _stage1_fin_kernel(hmax_ref, hsum_ref, hssq_ref, g_ref, b_ref, x_ref):
    m = jnp.float32(B * N * KNB)
    mean = jnp.sum(hsum_ref[:, 0, :], axis=0) / m
    ex2 = jnp.sum(hssq_ref[:, 0, :], axis=0) / m
    var = ex2 - mean * mean
    xn = (hmax_ref[...] - mean[None, :]) / jnp.sqrt(var + 1e-5)[None, :]
    h = xn * g_ref[0][None, :] + b_ref[0][None, :]
    x_ref[...] = jnp.where(h > 0, h, 0.2 * h)


def _stage1_fin(hmax, hsum, hssq, g, b):
    return pl.pallas_call(
        _stage1_fin_kernel,
        out_shape=jax.ShapeDtypeStruct((B * N, CH), jnp.float32),
    )(hmax.reshape(B * N, CH), hsum, hssq, g.reshape(1, CH), b.reshape(1, CH))


def _knn_stage_kernel(rows_ref, at_ref, dt_ref, idx_ref, u_ref, v_ref, s_scr):
    b = pl.program_id(0)
    rows = rows_ref[0]
    rb = rows.astype(jnp.bfloat16)
    s = _dot(rb, rb, (((1,), (1,))), _DEFAULT)
    xx = jnp.sum(rows * rows, axis=1)
    p = (-xx[:, None] + 2.0 * s) - xx[None, :]
    iota = lax.broadcasted_iota(jnp.int32, (N, N), 1)
    s_scr[...] = _pack_keys(p, iota)

    def body(k, m):
        kc = s_scr[...]
        mn = jnp.max(jnp.where(kc < m[:, None], kc, jnp.int32(-2 ** 31)),
                     axis=1)
        a = jnp.int32(N - 1) - (mn & jnp.int32(N - 1))
        idx_ref[0, pl.ds(k, 1), :] = (a + b * N)[None, :]
        return mn

    lax.fori_loop(0, KNB, body, jnp.full((N,), 2 ** 31 - 1, jnp.int32))
    u_ref[0] = _dot(rows, at_ref[...], (((1,), (0,))), _HIGHEST)
    v_ref[0] = _dot(rows, dt_ref[...], (((1,), (0,))), _HIGHEST)


def _knn_stage(rows, a_t, d_t):
    c = rows.shape[-1]
    return pl.pallas_call(
        _knn_stage_kernel,
        grid=(B,),
        in_specs=[
            pl.BlockSpec((1, N, c), lambda b: (b, 0, 0)),
            pl.BlockSpec((c, CHP), lambda b: (0, 0)),
            pl.BlockSpec((c, CH), lambda b: (0, 0)),
        ],
        out_specs=[
            pl.BlockSpec((1, KNB, N), lambda b: (b, 0, 0)),
            pl.BlockSpec((1, N, CHP), lambda b: (b, 0, 0)),
            pl.BlockSpec((1, N, CH), lambda b: (b, 0, 0)),
        ],
        out_shape=[
            jax.ShapeDtypeStruct((B, KNB, N), jnp.int32),
            jax.ShapeDtypeStruct((B, N, CHP), jnp.float32),
            jax.ShapeDtypeStruct((B, N, CH), jnp.float32),
        ],
        scratch_shapes=[pltpu.VMEM((N, N), jnp.int32)],
    )(rows, a_t, d_t)


_NWORK = 32
_PER_W = (B * N) // _NWORK
_NCK = 32
_NCHUNK = _PER_W // _NCK


def _sc_reduce_body(u_hbm, idx_hbm, max_hbm, sum_hbm, ssq_hbm,
                    idx_v, rows_v, omax_v, osum_v, ossq_v, sem):
    wid = lax.axis_index("s") * 2 + lax.axis_index("c")
    g0 = wid * _PER_W
    bb = g0 // N
    i0 = g0 % N

    cps = [pltpu.async_copy(idx_hbm.at[bb, k, pl.ds(i0, _PER_W)],
                            idx_v.at[k], sem) for k in range(KNB)]
    for cp in cps:
        cp.wait()

    for ck in range(_NCHUNK):
        cps = [pltpu.async_copy(
            u_hbm.at[idx_v.at[k, pl.ds(ck * _NCK, _NCK)]],
            rows_v.at[k], sem) for k in range(KNB)]
        for cp in cps:
            cp.wait()

        def body(il, _):
            for ch in range(CH // 16):
                sl = pl.ds(ch * 16, 16)
                r = rows_v[0, il, sl]
                vmax = r
                vsum = r
                vssq = r * r
                for k in range(1, KNB):
                    r = rows_v[k, il, sl]
                    vmax = jnp.maximum(vmax, r)
                    vsum = vsum + r
                    vssq = vssq + r * r
                omax_v[il, sl] = vmax
                osum_v[il, sl] = vsum
                ossq_v[il, sl] = vssq
            return 0

        lax.fori_loop(0, _NCK, body, 0)
        base = g0 + ck * _NCK
        pltpu.sync_copy(omax_v, max_hbm.at[pl.ds(base, _NCK)])
        pltpu.sync_copy(osum_v, sum_hbm.at[pl.ds(base, _NCK)])
        pltpu.sync_copy(ossq_v, ssq_hbm.at[pl.ds(base, _NCK)])


def _sc_reduce(u_flat, idx):
    mesh = plsc.VectorSubcoreMesh(core_axis_name="c", subcore_axis_name="s")
    shp = jax.ShapeDtypeStruct((B * N, CH), jnp.float32)
    fn = pl.kernel(
        _sc_reduce_body,
        mesh=mesh,
        out_type=[shp, shp, shp],
        scratch_types=[
            pltpu.VMEM((KNB, _PER_W), jnp.int32),
            pltpu.VMEM((KNB, _NCK, CHP), jnp.float32),
            pltpu.VMEM((_NCK, CH), jnp.float32),
            pltpu.VMEM((_NCK, CH), jnp.float32),
            pltpu.VMEM((_NCK, CH), jnp.float32),
            pltpu.SemaphoreType.DMA,
        ],
    )
    return fn(u_flat, idx)


def _stats_kernel(usum_ref, ussq_ref, v_ref, g_ref, b_ref, sc_ref, sh_ref):
    usum = usum_ref[...]
    ussq = ussq_ref[...]
    v = v_ref[...]
    m = jnp.float32(B * N * KNB)
    sum_s = jnp.sum(usum, axis=0)
    sum_v = jnp.sum(v, axis=0)
    sum_s2 = jnp.sum(ussq, axis=0)
    sum_vs = jnp.sum(v * usum, axis=0)
    sum_v2 = jnp.sum(v * v, axis=0)
    mean = (sum_s + KNB * sum_v) / m
    ex2 = (sum_s2 + 2.0 * sum_vs + KNB * sum_v2) / m
    var = ex2 - mean * mean
    sc = g_ref[0] * lax.rsqrt(var + 1e-5)
    sh = b_ref[0] - mean * sc
    sc_ref[...] = jnp.broadcast_to(sc[None, :], (8, CH))
    sh_ref[...] = jnp.broadcast_to(sh[None, :], (8, CH))


def _bn_stats(usum, ussq, v_flat, g, b):
    return pl.pallas_call(
        _stats_kernel,
        out_shape=[jax.ShapeDtypeStruct((8, CH), jnp.float32),
                   jax.ShapeDtypeStruct((8, CH), jnp.float32)],
    )(usum, ussq, v_flat, g.reshape(1, CH), b.reshape(1, CH))


def _finalize_kernel(umax_ref, v_ref, sc_ref, sh_ref, x_ref):
    h = (umax_ref[...] + v_ref[...]) * sc_ref[0] + sh_ref[0]
    x_ref[...] = jnp.where(h > 0, h, 0.2 * h)


def _finalize(umax, v_flat, sc, sh):
    return pl.pallas_call(
        _finalize_kernel,
        out_shape=jax.ShapeDtypeStruct((B * N, CH), jnp.float32),
    )(umax, v_flat, sc, sh)


def _head_kernel(x1_ref, x2_ref, w3t_ref, g3_ref, b3_ref, wfct_ref, bfc_ref,
                 wp1t_ref, bp1_ref, wp2t_ref, bp2_ref, rep_ref, proj_ref):
    xcomb = jnp.concatenate([x1_ref[...], x2_ref[...]], axis=1)
    m3 = jnp.float32(B * N)
    mu = jnp.sum(xcomb, axis=0) / m3
    m2 = _dot(xcomb, xcomb, (((0,), (0,))), _HIGHEST) / m3
    w3t = w3t_ref[...]
    mean3 = mu @ w3t
    e2 = jnp.sum((m2 @ w3t) * w3t, axis=0)
    var3 = e2 - mean3 * mean3
    h3 = _dot(xcomb, w3t, (((1,), (0,))), _DEFAULT)
    h3max = jnp.max(h3.reshape(B, N, 512), axis=1)
    sc3 = g3_ref[0] * lax.rsqrt(var3 + 1e-5)
    xo = (h3max - mean3[None, :]) * sc3[None, :] + b3_ref[0][None, :]
    xo = jnp.where(xo > 0, xo, 0.2 * xo)
    rep = _dot(xo, wfct_ref[...], (((1,), (0,))), _DEFAULT) + bfc_ref[0][None, :]
    p1 = jnp.maximum(
        _dot(rep, wp1t_ref[...], (((1,), (0,))), _DEFAULT) + bp1_ref[0][None, :],
        0.0)
    proj = _dot(p1, wp2t_ref[...], (((1,), (0,))), _DEFAULT) + bp2_ref[0][None, :]
    rep_ref[...] = rep
    proj_ref[...] = proj


def _head(x1_flat, x2_flat, w3, g3, b3, wfc, bfc, wp1, bp1, wp2, bp2):
    return pl.pallas_call(
        _head_kernel,
        out_shape=[jax.ShapeDtypeStruct((B, 512), jnp.float32),
                   jax.ShapeDtypeStruct((B, 32), jnp.float32)],
    )(x1_flat, x2_flat, w3.T, g3.reshape(1, 512), b3.reshape(1, 512),
      wfc.T, bfc.reshape(1, 512), wp1.T, bp1.reshape(1, 256),
      wp2.T, bp2.reshape(1, 32))


def _edge_stage(rows, w, g, b, cin):
    a_t = jnp.zeros((cin, CHP), jnp.float32).at[:, :CH].set(w[:, :cin].T)
    d_t = (w[:, cin:] - w[:, :cin]).T
    idx, u, v = _knn_stage(rows, a_t, d_t)
    u_flat = u.reshape(B * N, CHP)
    v_flat = v.reshape(B * N, CH)
    umax, usum, ussq = _sc_reduce(u_flat, idx)
    sc, sh = _bn_stats(usum, ussq, v_flat, g, b)
    return _finalize(umax, v_flat, sc, sh)


def kernel(x, W1, g1, b1, W2, g2, b2, W3, g3, b3, Wfc, bfc, Wp1, bp1, Wp2, bp2):
    xt = jnp.transpose(x, (0, 2, 1))
    hmax, hsum, hssq = _stage1(xt, W1.T)
    x1 = _stage1_fin(hmax, hsum, hssq, g1, b1)
    x2 = _edge_stage(x1.reshape(B, N, CH), W2, g2, b2, CH)
    return _head(x1, x2, W3, g3, b3, Wfc, bfc, Wp1, bp1, Wp2, bp2)


# speedup vs baseline: 1.0308x; 1.0136x over previous
import functools

import jax
import jax.numpy as jnp
from jax import lax
from jax.experimental import pallas as pl
from jax.experimental.pallas import tpu as pltpu
from jax.experimental.pallas import tpu_sc as plsc

B = 8
N = 1024
KNB = 20
CH = 64
CHP = 128

_DEFAULT = lax.Precision.DEFAULT
_HIGHEST = lax.Precision.HIGHEST


def _dot(a, b, dims, precision):
    return lax.dot_general(a, b, (dims, ((), ())), precision=precision,
                           preferred_element_type=jnp.float32)


def _pack_keys(p, iota):
    pi = lax.bitcast_convert_type(p, jnp.int32)
    pi = jnp.where(pi < 0, pi ^ jnp.int32(0x7FFFFFFF), pi)
    return (pi & jnp.int32(-1024)) | (jnp.int32(N - 1) - iota)


def _stage1_kernel(rows_ref, w1t_ref, hmax_ref, hsum_ref, hssq_ref, s_scr):
    rows = rows_ref[0]
    rb = rows.astype(jnp.bfloat16)
    s = _dot(rb, rb, (((1,), (1,))), _DEFAULT)
    xx = jnp.sum(rows * rows, axis=1)
    p = (-xx[:, None] + 2.0 * s) - xx[None, :]
    iota = lax.broadcasted_iota(jnp.int32, (N, N), 1)
    s_scr[...] = _pack_keys(p, iota)
    w1t = w1t_ref[...]
    h1 = rows.astype(jnp.bfloat16)
    r1 = rows - h1.astype(jnp.float32)
    h2 = r1.astype(jnp.bfloat16)
    r2 = r1 - h2.astype(jnp.float32)
    h3 = r2.astype(jnp.bfloat16)
    xcat = jnp.concatenate([h1, h2, h3], axis=1)

    def body(k, carry):
        hmax, hsum, hssq = carry
        kc = s_scr[...]
        m = jnp.max(kc, axis=1)
        a = jnp.int32(N - 1) - (m & jnp.int32(N - 1))
        sel = iota == a[:, None]
        s_scr[...] = jnp.where(sel, jnp.int32(-2 ** 31 + 1), kc)
        xjp = _dot(sel.astype(jnp.bfloat16), xcat, (((1,), (0,))), _DEFAULT)
        xj = (xjp[:, 0:3] + xjp[:, 3:6]) + xjp[:, 6:9]
        f6 = jnp.concatenate([xj - rows, rows], axis=1)
        hk = _dot(f6, w1t, (((1,), (0,))), _DEFAULT)
        return (jnp.maximum(hmax, hk), hsum + hk, hssq + hk * hk)

    hmax, hsum, hssq = lax.fori_loop(
        0, KNB, body,
        (jnp.full((N, CH), -3.0e38, jnp.float32),
         jnp.zeros((N, CH), jnp.float32),
         jnp.zeros((N, CH), jnp.float32)))
    hmax_ref[0] = hmax
    hsum_ref[0, 0] = jnp.sum(hsum, axis=0)
    hssq_ref[0, 0] = jnp.sum(hssq, axis=0)


def _stage1(rows, w1t):
    return pl.pallas_call(
        _stage1_kernel,
        grid=(B,),
        in_specs=[
            pl.BlockSpec((1, N, 3), lambda b: (b, 0, 0)),
            pl.BlockSpec((6, CH), lambda b: (0, 0)),
        ],
        out_specs=[
            pl.BlockSpec((1, N, CH), lambda b: (b, 0, 0)),
            pl.BlockSpec((1, 1, CH), lambda b: (b, 0, 0)),
            pl.BlockSpec((1, 1, CH), lambda b: (b, 0, 0)),
        ],
        out_shape=[
            jax.ShapeDtypeStruct((B, N, CH), jnp.float32),
            jax.ShapeDtypeStruct((B, 1, CH), jnp.float32),
            jax.ShapeDtypeStruct((B, 1, CH), jnp.float32),
        ],
        scratch_shapes=[pltpu.VMEM((N, N), jnp.int32)],
    )(rows, w1t)


def _stage1_fin_kernel(hmax_ref, hsum_ref, hssq_ref, g_ref, b_ref, x_ref):
    m = jnp.float32(B * N * KNB)
    mean = jnp.sum(hsum_ref[:, 0, :], axis=0) / m
    ex2 = jnp.sum(hssq_ref[:, 0, :], axis=0) / m
    var = ex2 - mean * mean
    xn = (hmax_ref[...] - mean[None, :]) / jnp.sqrt(var + 1e-5)[None, :]
    h = xn * g_ref[0][None, :] + b_ref[0][None, :]
    x_ref[...] = jnp.where(h > 0, h, 0.2 * h)


def _stage1_fin(hmax, hsum, hssq, g, b):
    return pl.pallas_call(
        _stage1_fin_kernel,
        out_shape=jax.ShapeDtypeStruct((B * N, CH), jnp.float32),
    )(hmax.reshape(B * N, CH), hsum, hssq, g.reshape(1, CH), b.reshape(1, CH))


def _knn_stage_kernel(rows_ref, at_ref, dt_ref, idx_ref, u_ref, v_ref, s_scr):
    b = pl.program_id(0)
    rows = rows_ref[0]
    rb = rows.astype(jnp.bfloat16)
    s = _dot(rb, rb, (((1,), (1,))), _DEFAULT)
    xx = jnp.sum(rows * rows, axis=1)
    p = (-xx[:, None] + 2.0 * s) - xx[None, :]
    iota = lax.broadcasted_iota(jnp.int32, (N, N), 1)
    s_scr[...] = _pack_keys(p, iota)

    def body(k, _):
        kc = s_scr[...]
        m = jnp.max(kc, axis=1)
        a = jnp.int32(N - 1) - (m & jnp.int32(N - 1))
        idx_ref[0, pl.ds(k, 1), :] = (a + b * N)[None, :]
        s_scr[...] = jnp.where(iota == a[:, None], jnp.int32(-2 ** 31 + 1), kc)
        return 0

    lax.fori_loop(0, KNB, body, 0)
    u_ref[0] = _dot(rows, at_ref[...], (((1,), (0,))), _HIGHEST)
    v_ref[0] = _dot(rows, dt_ref[...], (((1,), (0,))), _HIGHEST)


def _knn_stage(rows, a_t, d_t):
    c = rows.shape[-1]
    return pl.pallas_call(
        _knn_stage_kernel,
        grid=(B,),
        in_specs=[
            pl.BlockSpec((1, N, c), lambda b: (b, 0, 0)),
            pl.BlockSpec((c, CHP), lambda b: (0, 0)),
            pl.BlockSpec((c, CH), lambda b: (0, 0)),
        ],
        out_specs=[
            pl.BlockSpec((1, KNB, N), lambda b: (b, 0, 0)),
            pl.BlockSpec((1, N, CHP), lambda b: (b, 0, 0)),
            pl.BlockSpec((1, N, CH), lambda b: (b, 0, 0)),
        ],
        out_shape=[
            jax.ShapeDtypeStruct((B, KNB, N), jnp.int32),
            jax.ShapeDtypeStruct((B, N, CHP), jnp.float32),
            jax.ShapeDtypeStruct((B, N, CH), jnp.float32),
        ],
        scratch_shapes=[pltpu.VMEM((N, N), jnp.int32)],
    )(rows, a_t, d_t)


_NWORK = 32
_PER_W = (B * N) // _NWORK
_NCK = 32
_NCHUNK = _PER_W // _NCK


def _sc_reduce_body(u_hbm, idx_hbm, max_hbm, sum_hbm, ssq_hbm,
                    idx_v, rows_v, omax_v, osum_v, ossq_v, sem):
    wid = lax.axis_index("s") * 2 + lax.axis_index("c")
    g0 = wid * _PER_W
    bb = g0 // N
    i0 = g0 % N

    cps = [pltpu.async_copy(idx_hbm.at[bb, k, pl.ds(i0, _PER_W)],
                            idx_v.at[k], sem) for k in range(KNB)]
    for cp in cps:
        cp.wait()

    for ck in range(_NCHUNK):
        cps = [pltpu.async_copy(
            u_hbm.at[idx_v.at[k, pl.ds(ck * _NCK, _NCK)]],
            rows_v.at[k], sem) for k in range(KNB)]
        for cp in cps:
            cp.wait()

        def body(il, _):
            for ch in range(CH // 16):
                sl = pl.ds(ch * 16, 16)
                r = rows_v[0, il, sl]
                vmax = r
                vsum = r
                vssq = r * r
                for k in range(1, KNB):
                    r = rows_v[k, il, sl]
                    vmax = jnp.maximum(vmax, r)
                    vsum = vsum + r
                    vssq = vssq + r * r
                omax_v[il, sl] = vmax
                osum_v[il, sl] = vsum
                ossq_v[il, sl] = vssq
            return 0

        lax.fori_loop(0, _NCK, body, 0)
        base = g0 + ck * _NCK
        pltpu.sync_copy(omax_v, max_hbm.at[pl.ds(base, _NCK)])
        pltpu.sync_copy(osum_v, sum_hbm.at[pl.ds(base, _NCK)])
        pltpu.sync_copy(ossq_v, ssq_hbm.at[pl.ds(base, _NCK)])


def _sc_reduce(u_flat, idx):
    mesh = plsc.VectorSubcoreMesh(core_axis_name="c", subcore_axis_name="s")
    shp = jax.ShapeDtypeStruct((B * N, CH), jnp.float32)
    fn = pl.kernel(
        _sc_reduce_body,
        mesh=mesh,
        out_type=[shp, shp, shp],
        scratch_types=[
            pltpu.VMEM((KNB, _PER_W), jnp.int32),
            pltpu.VMEM((KNB, _NCK, CHP), jnp.float32),
            pltpu.VMEM((_NCK, CH), jnp.float32),
            pltpu.VMEM((_NCK, CH), jnp.float32),
            pltpu.VMEM((_NCK, CH), jnp.float32),
            pltpu.SemaphoreType.DMA,
        ],
    )
    return fn(u_flat, idx)


def _tail_kernel(umax_ref, usum_ref, ussq_ref, v_ref, g2_ref, b2_ref,
                 x1_ref, w3t_ref, g3_ref, b3_ref, wfct_ref, bfc_ref,
                 wp1t_ref, bp1_ref, wp2t_ref, bp2_ref, rep_ref, proj_ref):
    usum = usum_ref[...]
    ussq = ussq_ref[...]
    v = v_ref[...]
    m2n = jnp.float32(B * N * KNB)
    sum_s = jnp.sum(usum, axis=0)
    sum_v = jnp.sum(v, axis=0)
    sum_s2 = jnp.sum(ussq, axis=0)
    sum_vs = jnp.sum(v * usum, axis=0)
    sum_v2 = jnp.sum(v * v, axis=0)
    mean2 = (sum_s + KNB * sum_v) / m2n
    ex2_2 = (sum_s2 + 2.0 * sum_vs + KNB * sum_v2) / m2n
    var2 = ex2_2 - mean2 * mean2
    sc2 = g2_ref[0] * lax.rsqrt(var2 + 1e-5)
    sh2 = b2_ref[0] - mean2 * sc2
    h2 = (umax_ref[...] + v) * sc2[None, :] + sh2[None, :]
    x2 = jnp.where(h2 > 0, h2, 0.2 * h2)
    xcomb = jnp.concatenate([x1_ref[...], x2], axis=1)
    m3 = jnp.float32(B * N)
    mu = jnp.sum(xcomb, axis=0) / m3
    m2 = _dot(xcomb, xcomb, (((0,), (0,))), _HIGHEST) / m3
    w3t = w3t_ref[...]
    mean3 = mu @ w3t
    e2 = jnp.sum((m2 @ w3t) * w3t, axis=0)
    var3 = e2 - mean3 * mean3
    h3 = _dot(xcomb, w3t, (((1,), (0,))), _DEFAULT)
    h3max = jnp.max(h3.reshape(B, N, 512), axis=1)
    sc3 = g3_ref[0] * lax.rsqrt(var3 + 1e-5)
    xo = (h3max - mean3[None, :]) * sc3[None, :] + b3_ref[0][None, :]
    xo = jnp.where(xo > 0, xo, 0.2 * xo)
    rep = _dot(xo, wfct_ref[...], (((1,), (0,))), _DEFAULT) + bfc_ref[0][None, :]
    p1 = jnp.maximum(
        _dot(rep, wp1t_ref[...], (((1,), (0,))), _DEFAULT) + bp1_ref[0][None, :],
        0.0)
    proj = _dot(p1, wp2t_ref[...], (((1,), (0,))), _DEFAULT) + bp2_ref[0][None, :]
    rep_ref[...] = rep
    proj_ref[...] = proj


def _tail(umax, usum, ussq, v_flat, g2, b2, x1_flat,
          w3, g3, b3, wfc, bfc, wp1, bp1, wp2, bp2):
    return pl.pallas_call(
        _tail_kernel,
        out_shape=[jax.ShapeDtypeStruct((B, 512), jnp.float32),
                   jax.ShapeDtypeStruct((B, 32), jnp.float32)],
    )(umax, usum, ussq, v_flat, g2.reshape(1, CH), b2.reshape(1, CH),
      x1_flat, w3.T, g3.reshape(1, 512), b3.reshape(1, 512),
      wfc.T, bfc.reshape(1, 512), wp1.T, bp1.reshape(1, 256),
      wp2.T, bp2.reshape(1, 32))


def kernel(x, W1, g1, b1, W2, g2, b2, W3, g3, b3, Wfc, bfc, Wp1, bp1, Wp2, bp2):
    xt = jnp.transpose(x, (0, 2, 1))
    hmax, hsum, hssq = _stage1(xt, W1.T)
    x1 = _stage1_fin(hmax, hsum, hssq, g1, b1)
    a_t = jnp.zeros((CH, CHP), jnp.float32).at[:, :CH].set(W2[:, :CH].T)
    d_t = (W2[:, CH:] - W2[:, :CH]).T
    idx, u, v = _knn_stage(x1.reshape(B, N, CH), a_t, d_t)
    umax, usum, ussq = _sc_reduce(u.reshape(B * N, CHP), idx)
    return _tail(umax, usum, ussq, v.reshape(B * N, CH), g2, b2, x1,
                 W3, g3, b3, Wfc, bfc, Wp1, bp1, Wp2, bp2)
